# Initial kernel scaffold; baseline (speedup 1.0000x reference)
import jax
import jax.numpy as jnp
from jax.experimental import pallas as pl


def kernel(x, table):
    raise NotImplementedError("write your pallas kernel here")


# speedup vs baseline: 7.9383x; 7.9383x over previous
import functools

import jax
import jax.numpy as jnp
from jax import lax
from jax.experimental import pallas as pl
from jax.experimental.pallas import tpu as pltpu
from jax.experimental.pallas import tpu_sc as plsc

B = 4096
LH = 50
D = 64
NC = 2
NS = 16
NW = NC * NS
BPW = B // NW
C = 8
ROWS = C * LH
G = 80
GPC = ROWS // G
NCHUNK = BPW // C
IDXROWS = BPW * LH // G
LANES = 16
DV = D // LANES


def _make_sc_call():
    mesh = plsc.VectorSubcoreMesh(core_axis_name="c", subcore_axis_name="s")

    @functools.partial(
        pl.kernel,
        mesh=mesh,
        compiler_params=pltpu.CompilerParams(use_tc_tiling_on_sc=False),
        out_type=jax.ShapeDtypeStruct((B, D), jnp.float32),
        scratch_types=[
            pltpu.VMEM((IDXROWS, G), jnp.int32),
            pltpu.VMEM((ROWS, D), jnp.float32),
            pltpu.VMEM((C, D), jnp.float32),
            pltpu.SemaphoreType.DMA,
        ],
    )
    def sc_embed(x_hbm, tab_hbm, out_hbm, idx_v, rows_v, outc_v, sem):
        wid = lax.axis_index("s") * NC + lax.axis_index("c")
        pltpu.sync_copy(x_hbm.at[pl.ds(wid * IDXROWS, IDXROWS)], idx_v)

        def chunk(ci, carry):
            copies = []
            for j in range(GPC):
                copies.append(
                    pltpu.async_copy(
                        tab_hbm.at[idx_v.at[ci * GPC + j]],
                        rows_v.at[pl.ds(j * G, G)],
                        sem,
                    )
                )
            for cp in copies:
                cp.wait()

            for b in range(C):
                def body(l, acc):
                    r = b * LH + l
                    return tuple(
                        acc[d] + rows_v[r, pl.ds(d * LANES, LANES)]
                        for d in range(DV)
                    )

                acc0 = tuple(jnp.zeros((LANES,), jnp.float32) for _ in range(DV))
                acc = lax.fori_loop(0, LH, body, acc0)
                for d in range(DV):
                    outc_v[b, pl.ds(d * LANES, LANES)] = acc[d] * (1.0 / LH)

            pltpu.sync_copy(
                outc_v, out_hbm.at[pl.ds(wid * BPW + ci * C, C)]
            )
            return carry

        lax.fori_loop(0, NCHUNK, chunk, 0)

    return sc_embed


_sc_embed = _make_sc_call()


@jax.jit
def kernel(x, table):
    x2d = x.reshape(NW * IDXROWS, G)
    return _sc_embed(x2d, table)


# speedup vs baseline: 9.7474x; 1.2279x over previous
import functools

import jax
import jax.numpy as jnp
from jax import lax
from jax.experimental import pallas as pl
from jax.experimental.pallas import tpu as pltpu
from jax.experimental.pallas import tpu_sc as plsc

B = 4096
LH = 50
D = 64
NC = 2
NS = 16
NW = NC * NS
BPW = B // NW
C = 8
ROWS = C * LH
NCHUNK = BPW // C
LANES = 16
DV = D // LANES


def _make_sc_call():
    mesh = plsc.VectorSubcoreMesh(core_axis_name="c", subcore_axis_name="s")

    @functools.partial(
        pl.kernel,
        mesh=mesh,
        compiler_params=pltpu.CompilerParams(use_tc_tiling_on_sc=False),
        out_type=jax.ShapeDtypeStruct((B, D), jnp.float32),
        scratch_types=[
            pltpu.VMEM((BPW, LH), jnp.int32),
            pltpu.VMEM((ROWS, D), jnp.float32),
            pltpu.VMEM((ROWS, D), jnp.float32),
            pltpu.VMEM((C, D), jnp.float32),
            pltpu.SemaphoreType.DMA,
            pltpu.SemaphoreType.DMA,
        ],
    )
    def sc_embed(x_hbm, tab_hbm, out_hbm, idx_v, rows0, rows1, outc_v,
                 sem0, sem1):
        wid = lax.axis_index("s") * NC + lax.axis_index("c")
        pltpu.sync_copy(x_hbm.at[pl.ds(wid * BPW, BPW)], idx_v)

        def fire(ci, buf, sem):
            for j in range(C):
                pltpu.async_copy(
                    tab_hbm.at[idx_v.at[ci * C + j]],
                    buf.at[pl.ds(j * LH, LH)],
                    sem,
                )

        def drain(buf, sem):
            pltpu.make_async_copy(tab_hbm.at[pl.ds(0, ROWS)], buf, sem).wait()

        UNROLL = 5

        def accum_store(ci, buf):
            for b in range(C):
                def body(k, accs):
                    l = k * UNROLL
                    for u in range(UNROLL):
                        accs = tuple(
                            accs[d] + buf[b * LH + l + u,
                                          pl.ds(d * LANES, LANES)]
                            for d in range(DV)
                        )
                    return accs

                acc0 = tuple(
                    jnp.zeros((LANES,), jnp.float32) for _ in range(DV)
                )
                accs = lax.fori_loop(0, LH // UNROLL, body, acc0)
                for d in range(DV):
                    outc_v[b, pl.ds(d * LANES, LANES)] = accs[d] * (1.0 / LH)
            pltpu.sync_copy(outc_v, out_hbm.at[pl.ds(wid * BPW + ci * C, C)])

        fire(0, rows0, sem0)

        def body(i, carry):
            c0 = 2 * i
            c1 = 2 * i + 1
            fire(c1, rows1, sem1)
            drain(rows0, sem0)
            accum_store(c0, rows0)

            @pl.when(c1 + 1 < NCHUNK)
            def _():
                fire(c1 + 1, rows0, sem0)

            drain(rows1, sem1)
            accum_store(c1, rows1)
            return carry

        lax.fori_loop(0, NCHUNK // 2, body, 0)

    return sc_embed


_sc_embed = _make_sc_call()


@jax.jit
def kernel(x, table):
    return _sc_embed(x, table)
